# Initial kernel scaffold; baseline (speedup 1.0000x reference)
import jax
import jax.numpy as jnp
from jax.experimental import pallas as pl


def kernel(gw_state_real, gw_state_imag, prev_mem_real, prev_mem_imag, W_gate, b_gate, W_addr, b_addr, ln_w_r, ln_b_r, ln_w_i, ln_b_i):
    raise NotImplementedError("write your pallas kernel here")


# speedup vs baseline: 1.3350x; 1.3350x over previous
import jax
import jax.numpy as jnp
from jax.experimental import pallas as pl

_B = 1024
_DIM = 256
_SLOTS = 256
_TOPK = 3
_RB = 8


def _fused_kernel(qr_ref, qi_ref, mr_ref, mi_ref, wg_ref, bg_ref, wa_ref,
                  ba_ref, lnwr_ref, lnbr_ref, lnwi_ref, lnbi_ref,
                  read_r_ref, read_i_ref, next_r_ref, next_i_ref, ent_ref):
    qr = qr_ref[...]
    qi = qi_ref[...]
    mr = mr_ref[...]
    mi = mi_ref[...]

    sim = (mr * qr[:, None, :] + mi * qi[:, None, :]).sum(axis=-1)
    sim = sim - sim.max(axis=-1, keepdims=True)
    es = jnp.exp(sim)
    attn = (es / es.sum(axis=-1, keepdims=True))[:, :, None]
    read_r_ref[...] = (attn * mr).sum(axis=1)
    read_i_ref[...] = (attn * mi).sum(axis=1)

    flat = jnp.concatenate([qr, qi], axis=-1)
    gate = jax.nn.sigmoid(
        jnp.dot(flat, wg_ref[...], preferred_element_type=jnp.float32)
        + bg_ref[...])
    logits = jnp.dot(flat, wa_ref[...], preferred_element_type=jnp.float32)
    logits = logits + ba_ref[...]
    logits = logits - logits.max(axis=-1, keepdims=True)
    el = jnp.exp(logits)
    ww = el / el.sum(axis=-1, keepdims=True)

    ent_partial = -(ww * jnp.log(ww + 1e-10)).sum().reshape(1, 1)
    i = pl.program_id(0)

    @pl.when(i == 0)
    def _():
        ent_ref[...] = jnp.zeros_like(ent_ref)

    ent_ref[...] += ent_partial

    @pl.when(i == pl.num_programs(0) - 1)
    def _():
        ent_ref[...] *= (1.0 / _B)

    col = jax.lax.broadcasted_iota(jnp.int32, ww.shape, 1)
    w_work = ww
    sparse = jnp.zeros_like(ww)
    for _ in range(_TOPK):
        m = w_work.max(axis=-1, keepdims=True)
        at_max = w_work == m
        idx = jnp.min(jnp.where(at_max, col, _SLOTS), axis=-1, keepdims=True)
        onehot = col == idx
        sparse = jnp.where(onehot, ww, sparse)
        w_work = jnp.where(onehot, -jnp.inf, w_work)
    sparse = sparse / (sparse.sum(axis=-1, keepdims=True) + 1e-6)

    u = (gate * sparse)[:, :, None]
    nr = (1.0 - u) * mr + u * qr[:, None, :]
    ni = (1.0 - u) * mi + u * qi[:, None, :]

    mu_r = nr.mean(axis=-1, keepdims=True)
    var_r = ((nr - mu_r) ** 2).mean(axis=-1, keepdims=True)
    next_r_ref[...] = ((nr - mu_r) / jnp.sqrt(var_r + 1e-6)
                       * lnwr_ref[...] + lnbr_ref[...])
    mu_i = ni.mean(axis=-1, keepdims=True)
    var_i = ((ni - mu_i) ** 2).mean(axis=-1, keepdims=True)
    next_i_ref[...] = ((ni - mu_i) / jnp.sqrt(var_i + 1e-6)
                       * lnwi_ref[...] + lnbi_ref[...])


def kernel(gw_state_real, gw_state_imag, prev_mem_real, prev_mem_imag,
           W_gate, b_gate, W_addr, b_addr, ln_w_r, ln_b_r, ln_w_i, ln_b_i):
    grid = _B // _RB
    row_spec = pl.BlockSpec((_RB, _DIM), lambda i: (i, 0))
    mem_spec = pl.BlockSpec((_RB, _SLOTS, _DIM), lambda i: (i, 0, 0))
    full2 = lambda shape: pl.BlockSpec(shape, lambda i: (0, 0))

    out_shapes = (
        jax.ShapeDtypeStruct((_B, _DIM), jnp.float32),
        jax.ShapeDtypeStruct((_B, _DIM), jnp.float32),
        jax.ShapeDtypeStruct((_B, _SLOTS, _DIM), jnp.float32),
        jax.ShapeDtypeStruct((_B, _SLOTS, _DIM), jnp.float32),
        jax.ShapeDtypeStruct((1, 1), jnp.float32),
    )
    out_specs = (row_spec, row_spec, mem_spec, mem_spec, full2((1, 1)))

    in_specs = (
        row_spec, row_spec, mem_spec, mem_spec,
        full2((2 * _DIM, 1)),
        full2((1, 1)),
        full2((2 * _DIM, _SLOTS)),
        full2((1, _SLOTS)),
        full2((1, _DIM)),
        full2((1, _DIM)),
        full2((1, _DIM)),
        full2((1, _DIM)),
    )

    read_r, read_i, next_r, next_i, ent = pl.pallas_call(
        _fused_kernel,
        grid=(grid,),
        in_specs=list(in_specs),
        out_specs=list(out_specs),
        out_shape=out_shapes,
    )(gw_state_real, gw_state_imag, prev_mem_real, prev_mem_imag,
      W_gate, b_gate.reshape(1, 1), W_addr, b_addr.reshape(1, _SLOTS),
      ln_w_r.reshape(1, _DIM), ln_b_r.reshape(1, _DIM),
      ln_w_i.reshape(1, _DIM), ln_b_i.reshape(1, _DIM))

    return (read_r, read_i, next_r, next_i, ent[0, 0])


# speedup vs baseline: 1.5312x; 1.1470x over previous
import jax
import jax.numpy as jnp
from jax.experimental import pallas as pl

_B = 1024
_DIM = 256
_SLOTS = 256
_TOPK = 3
_RB = 16


def _fused_kernel(qr_ref, qi_ref, mr_ref, mi_ref, wg_ref, bg_ref, wa_ref,
                  ba_ref, lnwr_ref, lnbr_ref, lnwi_ref, lnbi_ref,
                  read_r_ref, read_i_ref, next_r_ref, next_i_ref, ent_ref):
    qr = qr_ref[...]
    qi = qi_ref[...]
    mr = mr_ref[...]
    mi = mi_ref[...]

    sim = (mr * qr[:, None, :] + mi * qi[:, None, :]).sum(axis=-1)
    sim = sim - sim.max(axis=-1, keepdims=True)
    es = jnp.exp(sim)
    attn = (es / es.sum(axis=-1, keepdims=True))[:, :, None]
    read_r_ref[...] = (attn * mr).sum(axis=1)
    read_i_ref[...] = (attn * mi).sum(axis=1)

    flat = jnp.concatenate([qr, qi], axis=-1)
    gate = jax.nn.sigmoid(
        jnp.dot(flat, wg_ref[...], preferred_element_type=jnp.float32)
        + bg_ref[...])
    logits = jnp.dot(flat, wa_ref[...], preferred_element_type=jnp.float32)
    logits = logits + ba_ref[...]
    logits = logits - logits.max(axis=-1, keepdims=True)
    el = jnp.exp(logits)
    ww = el / el.sum(axis=-1, keepdims=True)

    ent_partial = -(ww * jnp.log(ww + 1e-10)).sum().reshape(1, 1)
    i = pl.program_id(0)

    @pl.when(i == 0)
    def _():
        ent_ref[...] = jnp.zeros_like(ent_ref)

    ent_ref[...] += ent_partial

    @pl.when(i == pl.num_programs(0) - 1)
    def _():
        ent_ref[...] *= (1.0 / _B)

    col = jax.lax.broadcasted_iota(jnp.int32, ww.shape, 1)
    w_work = ww
    sparse = jnp.zeros_like(ww)
    for _ in range(_TOPK):
        m = w_work.max(axis=-1, keepdims=True)
        at_max = w_work == m
        idx = jnp.min(jnp.where(at_max, col, _SLOTS), axis=-1, keepdims=True)
        onehot = col == idx
        sparse = jnp.where(onehot, ww, sparse)
        w_work = jnp.where(onehot, -jnp.inf, w_work)
    sparse = sparse / (sparse.sum(axis=-1, keepdims=True) + 1e-6)

    u = (gate * sparse)[:, :, None]
    nr = (1.0 - u) * mr + u * qr[:, None, :]
    ni = (1.0 - u) * mi + u * qi[:, None, :]

    mu_r = nr.mean(axis=-1, keepdims=True)
    var_r = ((nr - mu_r) ** 2).mean(axis=-1, keepdims=True)
    next_r_ref[...] = ((nr - mu_r) / jnp.sqrt(var_r + 1e-6)
                       * lnwr_ref[...] + lnbr_ref[...])
    mu_i = ni.mean(axis=-1, keepdims=True)
    var_i = ((ni - mu_i) ** 2).mean(axis=-1, keepdims=True)
    next_i_ref[...] = ((ni - mu_i) / jnp.sqrt(var_i + 1e-6)
                       * lnwi_ref[...] + lnbi_ref[...])


def kernel(gw_state_real, gw_state_imag, prev_mem_real, prev_mem_imag,
           W_gate, b_gate, W_addr, b_addr, ln_w_r, ln_b_r, ln_w_i, ln_b_i):
    grid = _B // _RB
    row_spec = pl.BlockSpec((_RB, _DIM), lambda i: (i, 0))
    mem_spec = pl.BlockSpec((_RB, _SLOTS, _DIM), lambda i: (i, 0, 0))
    full2 = lambda shape: pl.BlockSpec(shape, lambda i: (0, 0))

    out_shapes = (
        jax.ShapeDtypeStruct((_B, _DIM), jnp.float32),
        jax.ShapeDtypeStruct((_B, _DIM), jnp.float32),
        jax.ShapeDtypeStruct((_B, _SLOTS, _DIM), jnp.float32),
        jax.ShapeDtypeStruct((_B, _SLOTS, _DIM), jnp.float32),
        jax.ShapeDtypeStruct((1, 1), jnp.float32),
    )
    out_specs = (row_spec, row_spec, mem_spec, mem_spec, full2((1, 1)))

    in_specs = (
        row_spec, row_spec, mem_spec, mem_spec,
        full2((2 * _DIM, 1)),
        full2((1, 1)),
        full2((2 * _DIM, _SLOTS)),
        full2((1, _SLOTS)),
        full2((1, _DIM)),
        full2((1, _DIM)),
        full2((1, _DIM)),
        full2((1, _DIM)),
    )

    read_r, read_i, next_r, next_i, ent = pl.pallas_call(
        _fused_kernel,
        grid=(grid,),
        in_specs=list(in_specs),
        out_specs=list(out_specs),
        out_shape=out_shapes,
    )(gw_state_real, gw_state_imag, prev_mem_real, prev_mem_imag,
      W_gate, b_gate.reshape(1, 1), W_addr, b_addr.reshape(1, _SLOTS),
      ln_w_r.reshape(1, _DIM), ln_b_r.reshape(1, _DIM),
      ln_w_i.reshape(1, _DIM), ln_b_i.reshape(1, _DIM))

    return (read_r, read_i, next_r, next_i, ent[0, 0])
